# baseline (speedup 1.0000x reference)
import functools

import jax
import jax.numpy as jnp
from jax import lax
from jax.experimental import pallas as pl
from jax.experimental.pallas import tpu as pltpu
from jax.experimental.pallas import tpu_sc as plsc

_B, _N, _K, _C = 4, 16384, 1024, 64
_S = 32
_R = 0.1
_R2 = _R * _R
_NVEC = _N // 16
_KSUB = _K // 8
_U = 16

_MESH = plsc.VectorSubcoreMesh(core_axis_name="c", subcore_axis_name="s")
_CPARAMS = pltpu.CompilerParams(needs_layout_passes=False)


def _wid():
    return lax.axis_index("s") * 2 + lax.axis_index("c")


def _ball_query_kernel(pts_hbm, cent_hbm, idx_out, gp_out, pts_v, cent_v,
                       idxb_v, gp_v, raw_v, craw_v):
    w = _wid()
    b = w // 8
    kgrp = w % 8
    koff = kgrp * _KSUB

    pltpu.sync_copy(pts_hbm.at[pl.ds(b * 3 * _N, 3 * _N)], raw_v)
    pltpu.sync_copy(cent_hbm.at[pl.ds((b * _K + koff) * 3, _KSUB * 3)],
                    craw_v)

    io16 = lax.iota(jnp.int32, 16)
    io48 = io16 * 3

    def deint_pts(v, _):
        base = pl.multiple_of(v * 16, 16)
        src = base * 3 + io48
        for c in range(3):
            pts_v[pl.ds(base + c * _N, 16)] = plsc.load_gather(
                raw_v, [src + c])
        return 0

    lax.fori_loop(0, _NVEC, deint_pts, 0)

    for v in range(_KSUB // 16):
        src = v * 48 + io48
        for c in range(3):
            cent_v[pl.ds(v * 16 + c * _KSUB, 16)] = plsc.load_gather(
                craw_v, [src + c])

    def per_centroid(k, _):
        kidx = jnp.full((16,), k, jnp.int32)
        cx = plsc.load_gather(cent_v, [kidx])
        cy = plsc.load_gather(cent_v, [kidx + _KSUB])
        cz = plsc.load_gather(cent_v, [kidx + 2 * _KSUB])
        row = pl.multiple_of(k * _S, 16)

        def cond(st):
            j, cnt = st
            return (j < _NVEC) & (cnt < _S)

        def body(st):
            j, cnt = st
            base0 = pl.multiple_of(j * 16, 16)
            ds = []
            for u in range(_U):
                base = base0 + u * 16
                dx = cx - pts_v[pl.ds(base, 16)]
                dy = cy - pts_v[pl.ds(base + _N, 16)]
                dz = cz - pts_v[pl.ds(base + 2 * _N, 16)]
                ds.append(dx * dx + dy * dy + dz * dz)
            dmin = ds[0]
            for u in range(1, _U):
                dmin = jnp.minimum(dmin, ds[u])
            t = plsc.all_reduce_population_count(
                dmin < jnp.float32(_R2))[0]

            def do_append():
                off = jnp.broadcast_to(cnt, (16,))
                for u in range(_U):
                    m = ds[u] < jnp.float32(_R2)
                    incl = plsc.cumsum(m.astype(jnp.int32))
                    pos = off + incl - 1
                    plsc.store_scatter(idxb_v, [row + pos],
                                       base0 + u * 16 + io16,
                                       mask=m & (pos < _S))
                    off = off + plsc.all_reduce_population_count(m)
                return off[0]

            cnt = lax.cond(t > 0, do_append, lambda: cnt)
            return (j + jnp.int32(_U), cnt)

        _, cnt = lax.while_loop(cond, body, (jnp.int32(0), jnp.int32(0)))

        pad = jnp.where(cnt > 0, idxb_v[pl.ds(row, 16)][0], 0)
        padv = jnp.broadcast_to(pad, (16,))
        plsc.store_scatter(idxb_v, [row + io16], padv, mask=io16 >= cnt)
        plsc.store_scatter(idxb_v, [row + io16 + 16], padv,
                           mask=(io16 + 16) >= cnt)

        iv0 = idxb_v[pl.ds(row, 16)]
        iv1 = idxb_v[pl.ds(row + 16, 16)]
        for c, cc in ((0, cx), (1, cy), (2, cz)):
            for h, iv in ((0, iv0), (1, iv1)):
                g = plsc.load_gather(pts_v, [iv + c * _N])
                gp_v[pl.ds(pl.multiple_of((c * _KSUB + k) * _S + h * 16, 16),
                           16)] = (g - cc) / jnp.float32(_R)
        return 0

    lax.fori_loop(0, _KSUB, per_centroid, 0)

    pltpu.sync_copy(idxb_v, idx_out.at[pl.ds((b * _K + koff) * _S,
                                             _KSUB * _S)])
    for c in range(3):
        pltpu.sync_copy(
            gp_v.at[pl.ds(c * _KSUB * _S, _KSUB * _S)],
            gp_out.at[pl.ds(((b * 3 + c) * _K + koff) * _S, _KSUB * _S)])


def _group_feats_kernel(feat_hbm, idx_hbm, nf_out, idx_v, rows_v, outs_v,
                        rsems, osems):
    w = _wid()
    b = w // 8
    c0 = (w % 8) * (_C // 8)
    nchan = _C // 8
    half = _K // 2

    pltpu.sync_copy(idx_hbm.at[pl.ds(b * _K * _S, _K * _S)], idx_v)

    def row_copy(ci, slot):
        return pltpu.async_copy(
            feat_hbm.at[pl.ds((b * _C + c0 + ci) * _N, _N)],
            rows_v.at[pl.ds(slot * _N, _N)], rsems[slot])

    pending_row = row_copy(0, 0)
    pending_out = [None, None]
    for ci in range(nchan):
        slot = ci % 2
        pending_row.wait()
        if ci + 1 < nchan:
            pending_row = row_copy(ci + 1, 1 - slot)
        roff = slot * _N
        for h in range(2):
            if pending_out[h] is not None:
                pending_out[h].wait()
            ooff = h * half * _S

            def _gather(k, _):
                ks = pl.multiple_of(k * _S, 16)
                os = pl.multiple_of((k - h * half) * _S + ooff, 16)
                outs_v[pl.ds(os, 16)] = plsc.load_gather(
                    rows_v, [idx_v[pl.ds(ks, 16)] + roff])
                outs_v[pl.ds(os + 16, 16)] = plsc.load_gather(
                    rows_v, [idx_v[pl.ds(ks + 16, 16)] + roff])
                return 0

            lax.fori_loop(h * half, (h + 1) * half, _gather, 0, unroll=4)

            pending_out[h] = pltpu.async_copy(
                outs_v.at[pl.ds(ooff, half * _S)],
                nf_out.at[pl.ds((b * _C + c0 + ci) * _K * _S + h * half * _S,
                                half * _S)],
                osems[h])
    for h in range(2):
        pending_out[h].wait()


@jax.jit
def kernel(points, centroids, features):
    pts_t = points.reshape(-1)
    cent_t = centroids.reshape(-1)

    ball = pl.kernel(
        _ball_query_kernel,
        mesh=_MESH,
        compiler_params=_CPARAMS,
        out_type=(
            jax.ShapeDtypeStruct((_B * _K * _S,), jnp.int32),
            jax.ShapeDtypeStruct((_B * 3 * _K * _S,), jnp.float32),
        ),
        scratch_types=[
            pltpu.VMEM((3 * _N,), jnp.float32),
            pltpu.VMEM((3 * _KSUB,), jnp.float32),
            pltpu.VMEM((_KSUB * _S,), jnp.int32),
            pltpu.VMEM((3 * _KSUB * _S,), jnp.float32),
            pltpu.VMEM((3 * _N,), jnp.float32),
            pltpu.VMEM((3 * _KSUB,), jnp.float32),
        ],
    )
    idx, gp = ball(pts_t, cent_t)
    grouped_pts = gp.reshape(_B, 3, _K, _S)

    group = pl.kernel(
        _group_feats_kernel,
        mesh=_MESH,
        compiler_params=_CPARAMS,
        out_type=jax.ShapeDtypeStruct((_B * _C * _K * _S,), jnp.float32),
        scratch_types=[
            pltpu.VMEM((_K * _S,), jnp.int32),
            pltpu.VMEM((2 * _N,), jnp.float32),
            pltpu.VMEM((_K * _S,), jnp.float32),
            [pltpu.SemaphoreType.DMA, pltpu.SemaphoreType.DMA],
            [pltpu.SemaphoreType.DMA, pltpu.SemaphoreType.DMA],
        ],
    )
    new_feats = group(features.reshape(-1), idx).reshape(_B, _C, _K, _S)
    return (grouped_pts, new_feats)


# speedup vs baseline: 1.1484x; 1.1484x over previous
import functools

import jax
import jax.numpy as jnp
from jax import lax
from jax.experimental import pallas as pl
from jax.experimental.pallas import tpu as pltpu
from jax.experimental.pallas import tpu_sc as plsc

_B, _N, _K, _C = 4, 16384, 1024, 64
_S = 32
_R = 0.1
_R2 = _R * _R
_NVEC = _N // 16
_KSUB = _K // 8
_U = 16

_MESH = plsc.VectorSubcoreMesh(core_axis_name="c", subcore_axis_name="s")
_CPARAMS = pltpu.CompilerParams(needs_layout_passes=False,
                                disable_bounds_checks=True)


def _wid():
    return lax.axis_index("s") * 2 + lax.axis_index("c")


def _ball_query_kernel(pts_hbm, cent_hbm, idx_out, gp_out, pts_v, cent_v,
                       idxb_v, gp_v):
    w = _wid()
    b = w // 8
    kgrp = w % 8
    koff = kgrp * _KSUB

    pltpu.sync_copy(pts_hbm.at[pl.ds(b * 3 * _N, 3 * _N)], pts_v)
    for c in range(3):
        pltpu.sync_copy(
            cent_hbm.at[pl.ds(b * 3 * _K + c * _K + koff, _KSUB)],
            cent_v.at[pl.ds(c * _KSUB, _KSUB)])

    io16 = lax.iota(jnp.int32, 16)

    def per_centroid(k, _):
        kidx = jnp.full((16,), k, jnp.int32)
        cx = plsc.load_gather(cent_v, [kidx])
        cy = plsc.load_gather(cent_v, [kidx + _KSUB])
        cz = plsc.load_gather(cent_v, [kidx + 2 * _KSUB])
        row = pl.multiple_of(k * _S, 16)

        def cond(st):
            j, cnt = st
            return (j < _NVEC) & (cnt < _S)

        def body(st):
            j, cnt = st
            base0 = pl.multiple_of(j * 16, 16)
            ds = []
            for u in range(_U):
                base = base0 + u * 16
                dx = cx - pts_v[pl.ds(base, 16)]
                dy = cy - pts_v[pl.ds(base + _N, 16)]
                dz = cz - pts_v[pl.ds(base + 2 * _N, 16)]
                ds.append(dx * dx + dy * dy + dz * dz)
            dmin = ds[0]
            for u in range(1, _U):
                dmin = jnp.minimum(dmin, ds[u])
            t = plsc.all_reduce_population_count(
                dmin < jnp.float32(_R2))[0]

            def do_append():
                off = jnp.broadcast_to(cnt, (16,))
                for u in range(_U):
                    m = ds[u] < jnp.float32(_R2)
                    incl = plsc.cumsum(m.astype(jnp.int32))
                    pos = off + incl - 1
                    plsc.store_scatter(idxb_v, [row + pos],
                                       base0 + u * 16 + io16,
                                       mask=m & (pos < _S))
                    off = off + plsc.all_reduce_population_count(m)
                return off[0]

            cnt = lax.cond(t > 0, do_append, lambda: cnt)
            return (j + jnp.int32(_U), cnt)

        _, cnt = lax.while_loop(cond, body, (jnp.int32(0), jnp.int32(0)))

        pad = jnp.where(cnt > 0, idxb_v[pl.ds(row, 16)][0], 0)
        padv = jnp.broadcast_to(pad, (16,))
        plsc.store_scatter(idxb_v, [row + io16], padv, mask=io16 >= cnt)
        plsc.store_scatter(idxb_v, [row + io16 + 16], padv,
                           mask=(io16 + 16) >= cnt)

        iv0 = idxb_v[pl.ds(row, 16)]
        iv1 = idxb_v[pl.ds(row + 16, 16)]
        for c, cc in ((0, cx), (1, cy), (2, cz)):
            for h, iv in ((0, iv0), (1, iv1)):
                g = plsc.load_gather(pts_v, [iv + c * _N])
                gp_v[pl.ds(pl.multiple_of((c * _KSUB + k) * _S + h * 16, 16),
                           16)] = (g - cc) / jnp.float32(_R)
        return 0

    lax.fori_loop(0, _KSUB, per_centroid, 0)

    pltpu.sync_copy(idxb_v, idx_out.at[pl.ds((b * _K + koff) * _S,
                                             _KSUB * _S)])
    for c in range(3):
        pltpu.sync_copy(
            gp_v.at[pl.ds(c * _KSUB * _S, _KSUB * _S)],
            gp_out.at[pl.ds(((b * 3 + c) * _K + koff) * _S, _KSUB * _S)])


def _group_feats_kernel(feat_hbm, idx_hbm, nf_out, idx_v, rows_v, outs_v,
                        rsems, osems):
    w = _wid()
    b = w // 8
    c0 = (w % 8) * (_C // 8)
    nchan = _C // 8
    half = _K // 2

    pltpu.sync_copy(idx_hbm.at[pl.ds(b * _K * _S, _K * _S)], idx_v)

    def row_copy(ci, slot):
        return pltpu.async_copy(
            feat_hbm.at[pl.ds((b * _C + c0 + ci) * _N, _N)],
            rows_v.at[pl.ds(slot * _N, _N)], rsems[slot])

    pending_row = row_copy(0, 0)
    pending_out = [None, None]
    for ci in range(nchan):
        slot = ci % 2
        pending_row.wait()
        if ci + 1 < nchan:
            pending_row = row_copy(ci + 1, 1 - slot)
        roff = slot * _N
        for h in range(2):
            if pending_out[h] is not None:
                pending_out[h].wait()
            ooff = h * half * _S

            def _gather(k, _):
                ks = pl.multiple_of(k * _S, 16)
                os = pl.multiple_of((k - h * half) * _S + ooff, 16)
                outs_v[pl.ds(os, 16)] = plsc.load_gather(
                    rows_v, [idx_v[pl.ds(ks, 16)] + roff])
                outs_v[pl.ds(os + 16, 16)] = plsc.load_gather(
                    rows_v, [idx_v[pl.ds(ks + 16, 16)] + roff])
                return 0

            lax.fori_loop(h * half, (h + 1) * half, _gather, 0)

            pending_out[h] = pltpu.async_copy(
                outs_v.at[pl.ds(ooff, half * _S)],
                nf_out.at[pl.ds((b * _C + c0 + ci) * _K * _S + h * half * _S,
                                half * _S)],
                osems[h])
    for h in range(2):
        pending_out[h].wait()


@jax.jit
def kernel(points, centroids, features):
    pts_t = jnp.transpose(points, (0, 2, 1)).reshape(-1)
    cent_t = jnp.transpose(centroids, (0, 2, 1)).reshape(-1)

    ball = pl.kernel(
        _ball_query_kernel,
        mesh=_MESH,
        compiler_params=_CPARAMS,
        out_type=(
            jax.ShapeDtypeStruct((_B * _K * _S,), jnp.int32),
            jax.ShapeDtypeStruct((_B * 3 * _K * _S,), jnp.float32),
        ),
        scratch_types=[
            pltpu.VMEM((3 * _N,), jnp.float32),
            pltpu.VMEM((3 * _KSUB,), jnp.float32),
            pltpu.VMEM((_KSUB * _S,), jnp.int32),
            pltpu.VMEM((3 * _KSUB * _S,), jnp.float32),
        ],
    )
    idx, gp = ball(pts_t, cent_t)
    grouped_pts = gp.reshape(_B, 3, _K, _S)

    group = pl.kernel(
        _group_feats_kernel,
        mesh=_MESH,
        compiler_params=_CPARAMS,
        out_type=jax.ShapeDtypeStruct((_B * _C * _K * _S,), jnp.float32),
        scratch_types=[
            pltpu.VMEM((_K * _S,), jnp.int32),
            pltpu.VMEM((2 * _N,), jnp.float32),
            pltpu.VMEM((_K * _S,), jnp.float32),
            [pltpu.SemaphoreType.DMA, pltpu.SemaphoreType.DMA],
            [pltpu.SemaphoreType.DMA, pltpu.SemaphoreType.DMA],
        ],
    )
    new_feats = group(features.reshape(-1), idx).reshape(_B, _C, _K, _S)
    return (grouped_pts, new_feats)


# speedup vs baseline: 1.2404x; 1.0801x over previous
import functools

import jax
import jax.numpy as jnp
from jax import lax
from jax.experimental import pallas as pl
from jax.experimental.pallas import tpu as pltpu
from jax.experimental.pallas import tpu_sc as plsc

_B, _N, _K, _C = 4, 16384, 1024, 64
_S = 32
_R = 0.1
_R2 = _R * _R
_NVEC = _N // 16
_KSUB = _K // 8
_U = 16

_MESH = plsc.VectorSubcoreMesh(core_axis_name="c", subcore_axis_name="s")
_CPARAMS = pltpu.CompilerParams(needs_layout_passes=False,
                                disable_bounds_checks=True)


def _wid():
    return lax.axis_index("s") * 2 + lax.axis_index("c")


def _ball_query_kernel(pts_hbm, cent_hbm, idx_out, gp_out, pts_v, cent_v,
                       idxb_v, gp_v):
    w = _wid()
    b = w // 8
    kgrp = w % 8
    koff = kgrp * _KSUB

    pltpu.sync_copy(pts_hbm.at[pl.ds(b * 3 * _N, 3 * _N)], pts_v)
    for c in range(3):
        pltpu.sync_copy(
            cent_hbm.at[pl.ds(b * 3 * _K + c * _K + koff, _KSUB)],
            cent_v.at[pl.ds(c * _KSUB, _KSUB)])

    io16 = lax.iota(jnp.int32, 16)

    def per_centroid(k, _):
        kidx = jnp.full((16,), k, jnp.int32)
        cx = plsc.load_gather(cent_v, [kidx])
        cy = plsc.load_gather(cent_v, [kidx + _KSUB])
        cz = plsc.load_gather(cent_v, [kidx + 2 * _KSUB])
        row = pl.multiple_of(k * _S, 16)

        def cond(st):
            j, cnt = st
            return (j < _NVEC) & (cnt < _S)

        def body(st):
            j, cnt = st
            base0 = pl.multiple_of(j * 16, 16)
            ds = []
            for u in range(_U):
                base = base0 + u * 16
                dx = cx - pts_v[pl.ds(base, 16)]
                dy = cy - pts_v[pl.ds(base + _N, 16)]
                dz = cz - pts_v[pl.ds(base + 2 * _N, 16)]
                ds.append(dx * dx + dy * dy + dz * dz)
            dmin = ds[0]
            for u in range(1, _U):
                dmin = jnp.minimum(dmin, ds[u])
            t = plsc.all_reduce_population_count(
                dmin < jnp.float32(_R2))[0]

            def do_append():
                off = jnp.broadcast_to(cnt, (16,))
                for u in range(_U):
                    m = ds[u] < jnp.float32(_R2)
                    incl = plsc.cumsum(m.astype(jnp.int32))
                    pos = off + incl - 1
                    plsc.store_scatter(idxb_v, [row + pos],
                                       base0 + u * 16 + io16,
                                       mask=m & (pos < _S))
                    off = off + plsc.all_reduce_population_count(m)
                return off[0]

            cnt = lax.cond(t > 0, do_append, lambda: cnt)
            return (j + jnp.int32(_U), cnt)

        _, cnt = lax.while_loop(cond, body, (jnp.int32(0), jnp.int32(0)))

        pad = jnp.where(cnt > 0, idxb_v[pl.ds(row, 16)][0], 0)
        padv = jnp.broadcast_to(pad, (16,))
        plsc.store_scatter(idxb_v, [row + io16], padv, mask=io16 >= cnt)
        plsc.store_scatter(idxb_v, [row + io16 + 16], padv,
                           mask=(io16 + 16) >= cnt)

        iv0 = idxb_v[pl.ds(row, 16)]
        iv1 = idxb_v[pl.ds(row + 16, 16)]
        for c, cc in ((0, cx), (1, cy), (2, cz)):
            for h, iv in ((0, iv0), (1, iv1)):
                g = plsc.load_gather(pts_v, [iv + c * _N])
                gp_v[pl.ds(pl.multiple_of((c * _KSUB + k) * _S + h * 16, 16),
                           16)] = (g - cc) / jnp.float32(_R)
        return 0

    lax.fori_loop(0, _KSUB, per_centroid, 0)

    pltpu.sync_copy(idxb_v, idx_out.at[pl.ds((b * _K + koff) * _S,
                                             _KSUB * _S)])
    for c in range(3):
        pltpu.sync_copy(
            gp_v.at[pl.ds(c * _KSUB * _S, _KSUB * _S)],
            gp_out.at[pl.ds(((b * 3 + c) * _K + koff) * _S, _KSUB * _S)])


def _group_feats_kernel(feat_hbm, idx_hbm, nf_out, idx_v, rows_v, outs_v,
                        rsems, osems):
    w = _wid()
    b = w // 8
    c0 = (w % 8) * (_C // 8)
    nchan = _C // 8
    half = _K // 2

    pltpu.sync_copy(idx_hbm.at[pl.ds(b * _K * _S, _K * _S)], idx_v)

    def row_copy(ci, slot):
        return pltpu.async_copy(
            feat_hbm.at[pl.ds((b * _C + c0 + ci) * _N, _N)],
            rows_v.at[pl.ds(slot * _N, _N)], rsems[slot])

    pending_row = row_copy(0, 0)
    pending_out = [None, None]
    for ci in range(nchan):
        slot = ci % 2
        pending_row.wait()
        if ci + 1 < nchan:
            pending_row = row_copy(ci + 1, 1 - slot)
        roff = slot * _N
        for h in range(2):
            if pending_out[h] is not None:
                pending_out[h].wait()
            ooff = h * half * _S

            def _gather(i, _):
                ks = pl.multiple_of(i * 4 * 16, 16)
                ivs = [idx_v[pl.ds(ks + 16 * q, 16)] + roff
                       for q in range(4)]
                gs = [plsc.load_gather(rows_v, [iv]) for iv in ivs]
                for q in range(4):
                    outs_v[pl.ds(ks + 16 * q, 16)] = gs[q]
                return 0

            lax.fori_loop(h * half // 2, (h + 1) * half // 2, _gather, 0)

            pending_out[h] = pltpu.async_copy(
                outs_v.at[pl.ds(ooff, half * _S)],
                nf_out.at[pl.ds((b * _C + c0 + ci) * _K * _S + h * half * _S,
                                half * _S)],
                osems[h])
    for h in range(2):
        pending_out[h].wait()


@jax.jit
def kernel(points, centroids, features):
    pts_t = jnp.transpose(points, (0, 2, 1)).reshape(-1)
    cent_t = jnp.transpose(centroids, (0, 2, 1)).reshape(-1)

    ball = pl.kernel(
        _ball_query_kernel,
        mesh=_MESH,
        compiler_params=_CPARAMS,
        out_type=(
            jax.ShapeDtypeStruct((_B * _K * _S,), jnp.int32),
            jax.ShapeDtypeStruct((_B * 3 * _K * _S,), jnp.float32),
        ),
        scratch_types=[
            pltpu.VMEM((3 * _N,), jnp.float32),
            pltpu.VMEM((3 * _KSUB,), jnp.float32),
            pltpu.VMEM((_KSUB * _S,), jnp.int32),
            pltpu.VMEM((3 * _KSUB * _S,), jnp.float32),
        ],
    )
    idx, gp = ball(pts_t, cent_t)
    grouped_pts = gp.reshape(_B, 3, _K, _S)

    group = pl.kernel(
        _group_feats_kernel,
        mesh=_MESH,
        compiler_params=_CPARAMS,
        out_type=jax.ShapeDtypeStruct((_B * _C * _K * _S,), jnp.float32),
        scratch_types=[
            pltpu.VMEM((_K * _S,), jnp.int32),
            pltpu.VMEM((2 * _N,), jnp.float32),
            pltpu.VMEM((_K * _S,), jnp.float32),
            [pltpu.SemaphoreType.DMA, pltpu.SemaphoreType.DMA],
            [pltpu.SemaphoreType.DMA, pltpu.SemaphoreType.DMA],
        ],
    )
    new_feats = group(features.reshape(-1), idx).reshape(_B, _C, _K, _S)
    return (grouped_pts, new_feats)
